# baseline (speedup 1.0000x reference)
import jax
import jax.numpy as jnp
from jax import lax
from jax.experimental import pallas as pl
from jax.experimental.pallas import tpu as pltpu
from jax.experimental.pallas import tpu_sc as plsc

L = 16
NC, NS = 2, 16
NW = NC * NS
ROWS = 256
BEAM = 4
RPW = ROWS // NW
V = 100000
NV = V // L
K = 64
NBINS = 4096
CAP = 256
CAP_SAFE = CAP - L
EQCAP = 64
MININT_PY = -(2**31)


def _mono(b):
    m = jnp.int32(MININT_PY)
    return jnp.where(b >= 0, b, jnp.bitwise_xor(jnp.bitwise_not(b), m))


def _inv(key):
    m = jnp.int32(MININT_PY)
    return jnp.where(key >= 0, key, jnp.bitwise_not(jnp.bitwise_xor(key, m)))


def _scal(v):
    return jnp.max(v)


def _store1(ref, idx, val, lanes):
    plsc.store_scatter(ref, (jnp.broadcast_to(idx, (L,)),),
                       jnp.broadcast_to(val, (L,)), mask=(lanes == 0))


def _load1(ref, idx):
    return jnp.max(plsc.load_gather(ref, (jnp.broadcast_to(idx, (L,)),)))


def _body(lp_hbm, g_hbm, sv_hbm, outs_hbm, outi_hbm,
          data, hist, ckeys, cpos, ekeys, epos, skeys, spos,
          gbuf, svbuf, obuf_s, obuf_i, sem0, sem1):
    wid = lax.axis_index("c") * NS + lax.axis_index("s")
    base_row = wid * RPW
    lanes = lax.iota(jnp.int32, L)
    ones = jnp.ones((L,), jnp.int32)
    zeros = jnp.zeros((L,), jnp.int32)
    minvec = jnp.full((L,), MININT_PY, jnp.int32)

    pltpu.sync_copy(sv_hbm.at[pl.ds(base_row, RPW)], svbuf)

    def row_body(r, _):
        row = base_row + r
        pltpu.sync_copy(lp_hbm.at[row // BEAM, row % BEAM], data)
        pltpu.sync_copy(g_hbm.at[pl.ds(row * K, K)], gbuf)

        def clear(nch):
            def cb(i):
                hist[pl.ds(i * L, L)] = zeros
            plsc.parallel_loop(0, nch, 1, unroll=8)(cb)

        def find_bin(c0, nch):
            def wcond(st):
                j, c, cprev = st
                return (c < 64) & (j >= 0)

            def wbody(st):
                j, c, cprev = st
                h = hist[pl.ds(j * L, L)]
                return (j - 1, c + jnp.sum(h), c)

            j, c, cprev = lax.while_loop(
                wcond, wbody, (jnp.int32(nch - 1), c0, c0))
            chunk = j + 1
            h = hist[pl.ds(chunk * L, L)]
            rc = lax.rev(plsc.cumsum(lax.rev(h, (0,))), (0,))
            cum_ge = cprev + rc
            cnt = jnp.sum((cum_ge >= 64).astype(jnp.int32))
            b_lane = cnt - 1
            n_b = jnp.sum(jnp.where(lanes == b_lane, h, 0))
            cg = jnp.sum(jnp.where(lanes == b_lane, cum_ge, 0))
            return chunk * L + b_lane, cg - n_b, n_b

        def run_level0(st):
            pval, c_above, done = st
            clear(NBINS // L)

            def hb(i):
                bv = lax.bitcast_convert_type(data[pl.ds(i * L, L)], jnp.int32)
                u = bv >> 20
                bucket = jnp.where(bv >= 0, u, u ^ 0x7FF) + 2048
                plsc.addupdate_scatter(hist, (bucket,), ones)
            plsc.parallel_loop(0, NV, 1, unroll=32)(hb)
            b, c_above, n_b = find_bin(jnp.int32(0), NBINS // L)
            pval = (b - 2048) << 20
            done = (c_above + n_b) <= CAP_SAFE
            return (pval, c_above, done)

        def run_refine(st):
            pval, c_above, done, lvl = st
            clear(NBINS // L)
            psh = jnp.where(lvl == 1, 20, 8)
            bsh = jnp.where(lvl == 1, 8, 0)
            pref = pval >> psh

            def hb(i):
                key = _mono(lax.bitcast_convert_type(data[pl.ds(i * L, L)],
                                                     jnp.int32))
                inp = (key >> psh) == pref
                bucket = (key >> bsh) & 0xFFF
                plsc.addupdate_scatter(hist, (bucket,), ones, mask=inp)
            plsc.parallel_loop(0, NV, 1, unroll=16)(hb)
            b, c_above, n_b = find_bin(c_above, NBINS // L)
            pval = pval | (b << bsh)
            done = (c_above + n_b) <= CAP_SAFE
            return (pval, c_above, done, lvl + 1)

        st0 = (jnp.int32(0), jnp.int32(0), jnp.bool_(False))
        st0 = run_level0(st0)
        st = (st0[0], st0[1], st0[2], jnp.int32(1))
        st = lax.while_loop(lambda q: (~q[2]) & (q[3] <= 2), run_refine, st)
        pval, c_above, done, _lvl = st

        def collect_a(_):
            def cb(i, off):
                key = _mono(lax.bitcast_convert_type(data[pl.ds(i * L, L)],
                                                     jnp.int32))
                m = key >= pval
                offc = jnp.minimum(off[0], CAP - L)
                plsc.store_compressed(cpos.at[pl.ds(offc, L)], i * L + lanes,
                                      mask=m)
                return off + plsc.all_reduce_population_count(m)
            off = plsc.parallel_loop(0, NV, 1, unroll=16,
                                     carry=jnp.zeros((L,), jnp.int32))(cb)
            return off[0]

        def collect_b(_):
            def cb(i, st2):
                offa, offb = st2
                key = _mono(lax.bitcast_convert_type(data[pl.ds(i * L, L)],
                                                     jnp.int32))
                mg = key > pval
                me = key == pval
                oa = jnp.minimum(offa, 80)
                ob = jnp.minimum(offb, EQCAP)
                plsc.store_compressed(ckeys.at[pl.ds(oa, L)], key, mask=mg)
                plsc.store_compressed(cpos.at[pl.ds(oa, L)], i * L + lanes,
                                      mask=mg)
                plsc.store_compressed(ekeys.at[pl.ds(ob, L)], key, mask=me)
                plsc.store_compressed(epos.at[pl.ds(ob, L)], i * L + lanes,
                                      mask=me)
                return (offa + jnp.sum(mg.astype(jnp.int32)),
                        offb + jnp.sum(me.astype(jnp.int32)))
            offa, offb = lax.fori_loop(0, NV, cb,
                                       (jnp.int32(0), jnp.int32(0)))
            nb_eq = jnp.minimum(offb, jnp.int32(EQCAP))

            def ab(c, _):
                ke = ekeys[pl.ds(c * L, L)]
                pe = epos[pl.ds(c * L, L)]
                valid = (c * L + lanes) < nb_eq
                ckeys[pl.ds(offa + c * L, L)] = jnp.where(valid, ke, minvec)
                cpos[pl.ds(offa + c * L, L)] = pe
                return 0
            lax.fori_loop(0, EQCAP // L, ab, 0)
            return offa + nb_eq

        m_cnt = lax.cond(done, collect_a, collect_b, 0)

        def kg(c):
            valid = (c * L + lanes) < m_cnt
            pos_c = cpos[pl.ds(c * L, L)]
            kv = plsc.load_gather(data, (pos_c,), mask=valid)
            key_c = _mono(lax.bitcast_convert_type(kv, jnp.int32))
            ckeys[pl.ds(c * L, L)] = jnp.where(valid, key_c, minvec)
        plsc.parallel_loop(0, CAP // L, 1, unroll=4)(kg)

        def bmax(c, cm):
            m = jnp.max(ckeys[pl.ds(c * L, L)])
            return jnp.where(lanes == c, m, cm)
        CM = lax.fori_loop(0, CAP // L, bmax, minvec)

        def sel(t, cm):
            gmax = jnp.max(cm)
            csp = plsc.all_reduce_ffs(cm == gmax)
            ck = plsc.load_gather(ckeys, (csp * L + lanes,))
            lsp = plsc.all_reduce_ffs(ck == gmax)
            sel_idx = csp * L + lsp
            pos = plsc.load_gather(cpos, (sel_idx,))
            _store1(skeys, t, gmax, lanes)
            _store1(spos, t, pos, lanes)
            plsc.store_scatter(ckeys, (sel_idx,), minvec, mask=(lanes == 0))
            ck2 = jnp.where(lanes == lsp, minvec, ck)
            return jnp.where(lanes == csp, jnp.max(ck2), cm)
        lax.fori_loop(0, K, sel, CM)

        sv0 = _inv(skeys[pl.ds(0, L)])
        sv1 = _inv(skeys[pl.ds(L, L)])
        sv2 = _inv(skeys[pl.ds(2 * L, L)])
        sv3 = _inv(skeys[pl.ds(3 * L, L)])
        s0 = lax.bitcast_convert_type(sv0, jnp.float32) + gbuf[pl.ds(0, L)]
        s1 = lax.bitcast_convert_type(sv1, jnp.float32) + gbuf[pl.ds(L, L)]
        s2 = lax.bitcast_convert_type(sv2, jnp.float32) + gbuf[pl.ds(2 * L, L)]
        s3 = lax.bitcast_convert_type(sv3, jnp.float32) + gbuf[pl.ds(3 * L, L)]
        m0, m1, m2, m3 = (jnp.max(s0), jnp.max(s1), jnp.max(s2), jnp.max(s3))
        gm = jnp.maximum(jnp.maximum(m0, m1), jnp.maximum(m2, m3))
        csel = jnp.where(m0 == gm, 0,
                         jnp.where(m1 == gm, 1, jnp.where(m2 == gm, 2, 3)))
        skc = skeys[pl.ds(csel * L, L)]
        scq = lax.bitcast_convert_type(_inv(skc), jnp.float32) + gbuf[pl.ds(csel * L, L)]
        lsel = _scal(plsc.all_reduce_ffs(scq == gm))
        val = jnp.sum(jnp.where(lanes == lsel,
                                lax.bitcast_convert_type(_inv(skc), jnp.float32), 0.0))
        vid = jnp.sum(jnp.where(lanes == lsel, spos[pl.ds(csel * L, L)], 0))
        _store1(obuf_s, r, val + _load1(svbuf, r), lanes)
        _store1(obuf_i, r, vid, lanes)
        return 0

    lax.fori_loop(0, RPW, row_body, 0)
    pltpu.sync_copy(obuf_s, outs_hbm.at[pl.ds(base_row, RPW)])
    pltpu.sync_copy(obuf_i, outi_hbm.at[pl.ds(base_row, RPW)])


def kernel(step, lprobs, scores):
    bsz, beam, vocab = lprobs.shape
    g = jax.random.gumbel(jax.random.key(42), (bsz * beam, K),
                          jnp.float32).reshape(-1)
    sv = lax.dynamic_index_in_dim(scores, step - 1, axis=2,
                                  keepdims=False).reshape(-1)

    mesh = plsc.VectorSubcoreMesh(core_axis_name="c", subcore_axis_name="s",
                                  num_cores=NC, num_subcores=NS)
    out_s, out_i = pl.kernel(
        _body,
        out_type=(jax.ShapeDtypeStruct((ROWS,), jnp.float32),
                  jax.ShapeDtypeStruct((ROWS,), jnp.int32)),
        mesh=mesh,
        compiler_params=pltpu.CompilerParams(needs_layout_passes=False),
        scratch_types=[
            pltpu.VMEM((V,), jnp.float32),
            pltpu.VMEM((NBINS,), jnp.int32),
            pltpu.VMEM((CAP,), jnp.int32),
            pltpu.VMEM((CAP,), jnp.int32),
            pltpu.VMEM((96,), jnp.int32),
            pltpu.VMEM((96,), jnp.int32),
            pltpu.VMEM((K,), jnp.int32),
            pltpu.VMEM((K,), jnp.int32),
            pltpu.VMEM((K,), jnp.float32),
            pltpu.VMEM((RPW,), jnp.float32),
            pltpu.VMEM((RPW,), jnp.float32),
            pltpu.VMEM((RPW,), jnp.int32),
            pltpu.SemaphoreType.DMA,
            pltpu.SemaphoreType.DMA,
        ],
    )(lprobs, g, sv)

    scores_buf = out_s.reshape(bsz, beam)
    indices_buf = out_i.reshape(bsz, beam)
    beams_buf = jnp.broadcast_to(jnp.arange(beam, dtype=jnp.int32),
                                 (bsz, beam))
    return scores_buf, indices_buf, beams_buf


# speedup vs baseline: 1.5066x; 1.5066x over previous
import jax
import jax.numpy as jnp
from jax import lax
from jax.experimental import pallas as pl
from jax.experimental.pallas import tpu as pltpu
from jax.experimental.pallas import tpu_sc as plsc

L = 16
NC, NS = 2, 16
NW = NC * NS
ROWS = 256
BEAM = 4
RPW = ROWS // NW
V = 100000
NV = V // L
K = 64
NBINS = 4096
CAP = 256
CAP_SAFE = CAP - L
EQCAP = 64
MININT_PY = -(2**31)


def _mono(b):
    m = jnp.int32(MININT_PY)
    return jnp.where(b >= 0, b, jnp.bitwise_xor(jnp.bitwise_not(b), m))


def _inv(key):
    m = jnp.int32(MININT_PY)
    return jnp.where(key >= 0, key, jnp.bitwise_not(jnp.bitwise_xor(key, m)))


def _scal(v):
    return jnp.max(v)


def _store1(ref, idx, val, lanes):
    plsc.store_scatter(ref, (jnp.broadcast_to(idx, (L,)),),
                       jnp.broadcast_to(val, (L,)), mask=(lanes == 0))


def _load1(ref, idx):
    return jnp.max(plsc.load_gather(ref, (jnp.broadcast_to(idx, (L,)),)))


def _body(lp_hbm, g_hbm, sv_hbm, outs_hbm, outi_hbm,
          data, hist, ckeys, cpos, ekeys, epos, skeys, spos,
          gbuf, svbuf, obuf_s, obuf_i, sem0, sem1):
    wid = lax.axis_index("c") * NS + lax.axis_index("s")
    base_row = wid * RPW
    lanes = lax.iota(jnp.int32, L)
    ones = jnp.ones((L,), jnp.int32)
    zeros = jnp.zeros((L,), jnp.int32)
    minvec = jnp.full((L,), MININT_PY, jnp.int32)

    pltpu.sync_copy(sv_hbm.at[pl.ds(base_row, RPW)], svbuf)

    def row_body(r, _):
        row = base_row + r
        pltpu.sync_copy(lp_hbm.at[row // BEAM, row % BEAM], data)
        pltpu.sync_copy(g_hbm.at[pl.ds(row * K, K)], gbuf)

        def clear(nch):
            def cb(i):
                hist[pl.ds(i * L, L)] = zeros
            plsc.parallel_loop(0, nch, 1, unroll=8)(cb)

        def find_bin(c0, nch):
            def wcond(st):
                j, c, cprev = st
                return (c < 64) & (j >= 0)

            def wbody(st):
                j, c, cprev = st
                h = hist[pl.ds(j * L, L)]
                return (j - 1, c + jnp.sum(h), c)

            j, c, cprev = lax.while_loop(
                wcond, wbody, (jnp.int32(nch - 1), c0, c0))
            chunk = j + 1
            h = hist[pl.ds(chunk * L, L)]
            rc = lax.rev(plsc.cumsum(lax.rev(h, (0,))), (0,))
            cum_ge = cprev + rc
            cnt = jnp.sum((cum_ge >= 64).astype(jnp.int32))
            b_lane = cnt - 1
            n_b = jnp.sum(jnp.where(lanes == b_lane, h, 0))
            cg = jnp.sum(jnp.where(lanes == b_lane, cum_ge, 0))
            return chunk * L + b_lane, cg - n_b, n_b

        def run_level0(st):
            pval, c_above, done = st
            clear(NBINS // L)

            def hb(i):
                bv = lax.bitcast_convert_type(data[pl.ds(i * L, L)], jnp.int32)
                u = bv >> 20
                bucket = jnp.where(bv >= 0, u, u ^ 0x7FF) + 2048
                plsc.addupdate_scatter(hist, (bucket,), ones)
            plsc.parallel_loop(0, NV, 1, unroll=16)(hb)
            b, c_above, n_b = find_bin(jnp.int32(0), NBINS // L)
            pval = (b - 2048) << 20
            done = (c_above + n_b) <= CAP_SAFE
            return (pval, c_above, done)

        def run_refine(st):
            pval, c_above, done, lvl = st
            clear(NBINS // L)
            psh = jnp.where(lvl == 1, 20, 8)
            bsh = jnp.where(lvl == 1, 8, 0)
            pref = pval >> psh

            def hb(i):
                key = _mono(lax.bitcast_convert_type(data[pl.ds(i * L, L)],
                                                     jnp.int32))
                inp = (key >> psh) == pref
                bucket = (key >> bsh) & 0xFFF
                plsc.addupdate_scatter(hist, (bucket,), ones, mask=inp)
            plsc.parallel_loop(0, NV, 1, unroll=16)(hb)
            b, c_above, n_b = find_bin(c_above, NBINS // L)
            pval = pval | (b << bsh)
            done = (c_above + n_b) <= CAP_SAFE
            return (pval, c_above, done, lvl + 1)

        st0 = (jnp.int32(0), jnp.int32(0), jnp.bool_(False))
        st0 = run_level0(st0)
        st = (st0[0], st0[1], st0[2], jnp.int32(1))
        st = lax.while_loop(lambda q: (~q[2]) & (q[3] <= 2), run_refine, st)
        pval, c_above, done, _lvl = st

        def collect_a(_):
            def cb(i, off):
                key = _mono(lax.bitcast_convert_type(data[pl.ds(i * L, L)],
                                                     jnp.int32))
                m = key >= pval
                offc = jnp.minimum(off[0], CAP - L)
                plsc.store_compressed(cpos.at[pl.ds(offc, L)], i * L + lanes,
                                      mask=m)
                return off + plsc.all_reduce_population_count(m)
            off = plsc.parallel_loop(0, NV, 1, unroll=8,
                                     carry=jnp.zeros((L,), jnp.int32))(cb)
            return off[0]

        def collect_b(_):
            def cb(i, st2):
                offa, offb = st2
                key = _mono(lax.bitcast_convert_type(data[pl.ds(i * L, L)],
                                                     jnp.int32))
                mg = key > pval
                me = key == pval
                oa = jnp.minimum(offa, 80)
                ob = jnp.minimum(offb, EQCAP)
                plsc.store_compressed(ckeys.at[pl.ds(oa, L)], key, mask=mg)
                plsc.store_compressed(cpos.at[pl.ds(oa, L)], i * L + lanes,
                                      mask=mg)
                plsc.store_compressed(ekeys.at[pl.ds(ob, L)], key, mask=me)
                plsc.store_compressed(epos.at[pl.ds(ob, L)], i * L + lanes,
                                      mask=me)
                return (offa + jnp.sum(mg.astype(jnp.int32)),
                        offb + jnp.sum(me.astype(jnp.int32)))
            offa, offb = lax.fori_loop(0, NV, cb,
                                       (jnp.int32(0), jnp.int32(0)))
            nb_eq = jnp.minimum(offb, jnp.int32(EQCAP))

            def ab(c, _):
                ke = ekeys[pl.ds(c * L, L)]
                pe = epos[pl.ds(c * L, L)]
                valid = (c * L + lanes) < nb_eq
                ckeys[pl.ds(offa + c * L, L)] = jnp.where(valid, ke, minvec)
                cpos[pl.ds(offa + c * L, L)] = pe
                return 0
            lax.fori_loop(0, EQCAP // L, ab, 0)
            return offa + nb_eq

        m_cnt = lax.cond(done, collect_a, collect_b, 0)

        def kg(c):
            valid = (c * L + lanes) < m_cnt
            pos_c = cpos[pl.ds(c * L, L)]
            kv = plsc.load_gather(data, (pos_c,), mask=valid)
            key_c = _mono(lax.bitcast_convert_type(kv, jnp.int32))
            ckeys[pl.ds(c * L, L)] = jnp.where(valid, key_c, minvec)
        plsc.parallel_loop(0, CAP // L, 1, unroll=4)(kg)

        def bmax(c, cm):
            m = jnp.max(ckeys[pl.ds(c * L, L)])
            return jnp.where(lanes == c, m, cm)
        CM = lax.fori_loop(0, CAP // L, bmax, minvec)

        def sel(t, cm):
            gmax = jnp.max(cm)
            csp = plsc.all_reduce_ffs(cm == gmax)
            ck = plsc.load_gather(ckeys, (csp * L + lanes,))
            lsp = plsc.all_reduce_ffs(ck == gmax)
            sel_idx = csp * L + lsp
            pos = plsc.load_gather(cpos, (sel_idx,))
            _store1(skeys, t, gmax, lanes)
            _store1(spos, t, pos, lanes)
            plsc.store_scatter(ckeys, (sel_idx,), minvec, mask=(lanes == 0))
            ck2 = jnp.where(lanes == lsp, minvec, ck)
            return jnp.where(lanes == csp, jnp.max(ck2), cm)
        lax.fori_loop(0, K, sel, CM)

        sv0 = _inv(skeys[pl.ds(0, L)])
        sv1 = _inv(skeys[pl.ds(L, L)])
        sv2 = _inv(skeys[pl.ds(2 * L, L)])
        sv3 = _inv(skeys[pl.ds(3 * L, L)])
        s0 = lax.bitcast_convert_type(sv0, jnp.float32) + gbuf[pl.ds(0, L)]
        s1 = lax.bitcast_convert_type(sv1, jnp.float32) + gbuf[pl.ds(L, L)]
        s2 = lax.bitcast_convert_type(sv2, jnp.float32) + gbuf[pl.ds(2 * L, L)]
        s3 = lax.bitcast_convert_type(sv3, jnp.float32) + gbuf[pl.ds(3 * L, L)]
        m0, m1, m2, m3 = (jnp.max(s0), jnp.max(s1), jnp.max(s2), jnp.max(s3))
        gm = jnp.maximum(jnp.maximum(m0, m1), jnp.maximum(m2, m3))
        csel = jnp.where(m0 == gm, 0,
                         jnp.where(m1 == gm, 1, jnp.where(m2 == gm, 2, 3)))
        skc = skeys[pl.ds(csel * L, L)]
        scq = lax.bitcast_convert_type(_inv(skc), jnp.float32) + gbuf[pl.ds(csel * L, L)]
        lsel = _scal(plsc.all_reduce_ffs(scq == gm))
        val = jnp.sum(jnp.where(lanes == lsel,
                                lax.bitcast_convert_type(_inv(skc), jnp.float32), 0.0))
        vid = jnp.sum(jnp.where(lanes == lsel, spos[pl.ds(csel * L, L)], 0))
        _store1(obuf_s, r, val + _load1(svbuf, r), lanes)
        _store1(obuf_i, r, vid, lanes)
        return 0

    lax.fori_loop(0, RPW, row_body, 0)
    pltpu.sync_copy(obuf_s, outs_hbm.at[pl.ds(base_row, RPW)])
    pltpu.sync_copy(obuf_i, outi_hbm.at[pl.ds(base_row, RPW)])


def kernel(step, lprobs, scores):
    bsz, beam, vocab = lprobs.shape
    g = jax.random.gumbel(jax.random.key(42), (bsz * beam, K),
                          jnp.float32).reshape(-1)
    sv = lax.dynamic_index_in_dim(scores, step - 1, axis=2,
                                  keepdims=False).reshape(-1)

    mesh = plsc.VectorSubcoreMesh(core_axis_name="c", subcore_axis_name="s",
                                  num_cores=NC, num_subcores=NS)
    out_s, out_i = pl.kernel(
        _body,
        out_type=(jax.ShapeDtypeStruct((ROWS,), jnp.float32),
                  jax.ShapeDtypeStruct((ROWS,), jnp.int32)),
        mesh=mesh,
        compiler_params=pltpu.CompilerParams(needs_layout_passes=False),
        scratch_types=[
            pltpu.VMEM((V,), jnp.float32),
            pltpu.VMEM((NBINS,), jnp.int32),
            pltpu.VMEM((CAP,), jnp.int32),
            pltpu.VMEM((CAP,), jnp.int32),
            pltpu.VMEM((96,), jnp.int32),
            pltpu.VMEM((96,), jnp.int32),
            pltpu.VMEM((K,), jnp.int32),
            pltpu.VMEM((K,), jnp.int32),
            pltpu.VMEM((K,), jnp.float32),
            pltpu.VMEM((RPW,), jnp.float32),
            pltpu.VMEM((RPW,), jnp.float32),
            pltpu.VMEM((RPW,), jnp.int32),
            pltpu.SemaphoreType.DMA,
            pltpu.SemaphoreType.DMA,
        ],
    )(lprobs, g, sv)

    scores_buf = out_s.reshape(bsz, beam)
    indices_buf = out_i.reshape(bsz, beam)
    beams_buf = jnp.broadcast_to(jnp.arange(beam, dtype=jnp.int32),
                                 (bsz, beam))
    return scores_buf, indices_buf, beams_buf


# speedup vs baseline: 1.6401x; 1.0886x over previous
import jax
import jax.numpy as jnp
from jax import lax
from jax.experimental import pallas as pl
from jax.experimental.pallas import tpu as pltpu
from jax.experimental.pallas import tpu_sc as plsc

L = 16
NC, NS = 2, 16
NW = NC * NS
ROWS = 256
BEAM = 4
RPW = ROWS // NW
V = 100000
NV = V // L
K = 64
NBINS = 4096
CAP = 256
CAP_SAFE = CAP - L
EQCAP = 64
MININT_PY = -(2**31)


def _mono(b):
    m = jnp.int32(MININT_PY)
    return jnp.where(b >= 0, b, jnp.bitwise_xor(jnp.bitwise_not(b), m))


def _inv(key):
    m = jnp.int32(MININT_PY)
    return jnp.where(key >= 0, key, jnp.bitwise_not(jnp.bitwise_xor(key, m)))


def _scal(v):
    return jnp.max(v)


def _store1(ref, idx, val, lanes):
    plsc.store_scatter(ref, (jnp.broadcast_to(idx, (L,)),),
                       jnp.broadcast_to(val, (L,)), mask=(lanes == 0))


def _load1(ref, idx):
    return jnp.max(plsc.load_gather(ref, (jnp.broadcast_to(idx, (L,)),)))


def _body(lp_hbm, g_hbm, sv_hbm, outs_hbm, outi_hbm,
          data, hist, ckeys, cpos, ekeys, epos, skeys, spos,
          gbuf, svbuf, obuf_s, obuf_i, sem0, sem1):
    wid = lax.axis_index("c") * NS + lax.axis_index("s")
    base_row = wid * RPW
    lanes = lax.iota(jnp.int32, L)
    ones = jnp.ones((L,), jnp.int32)
    zeros = jnp.zeros((L,), jnp.int32)
    minvec = jnp.full((L,), MININT_PY, jnp.int32)

    pltpu.sync_copy(sv_hbm.at[pl.ds(base_row, RPW)], svbuf)
    pltpu.async_copy(lp_hbm.at[base_row // BEAM, base_row % BEAM], data, sem0)

    def row_body(r, _):
        row = base_row + r
        pltpu.sync_copy(g_hbm.at[pl.ds(row * K, K)], gbuf)
        pltpu.make_async_copy(lp_hbm.at[row // BEAM, row % BEAM], data,
                              sem0).wait()

        def clear(nch):
            def cb(i):
                hist[pl.ds(i * L, L)] = zeros
            plsc.parallel_loop(0, nch, 1, unroll=8)(cb)

        def find_bin(c0, nch):
            def wcond(st):
                j, c, cprev = st
                return (c < 64) & (j >= 0)

            def wbody(st):
                j, c, cprev = st
                h = hist[pl.ds(j * L, L)]
                return (j - 1, c + jnp.sum(h), c)

            j, c, cprev = lax.while_loop(
                wcond, wbody, (jnp.int32(nch - 1), c0, c0))
            chunk = j + 1
            h = hist[pl.ds(chunk * L, L)]
            rc = lax.rev(plsc.cumsum(lax.rev(h, (0,))), (0,))
            cum_ge = cprev + rc
            cnt = jnp.sum((cum_ge >= 64).astype(jnp.int32))
            b_lane = cnt - 1
            n_b = jnp.sum(jnp.where(lanes == b_lane, h, 0))
            cg = jnp.sum(jnp.where(lanes == b_lane, cum_ge, 0))
            return chunk * L + b_lane, cg - n_b, n_b

        def run_level0(st):
            pval, c_above, done = st
            clear(NBINS // L)

            def hb(i):
                bv = lax.bitcast_convert_type(data[pl.ds(i * L, L)], jnp.int32)
                u = bv >> 20
                bucket = jnp.where(bv >= 0, u, u ^ 0x7FF) + 2048
                plsc.addupdate_scatter(hist, (bucket,), ones)
            plsc.parallel_loop(0, NV, 1, unroll=16)(hb)
            b, c_above, n_b = find_bin(jnp.int32(0), NBINS // L)
            pval = (b - 2048) << 20
            done = (c_above + n_b) <= CAP_SAFE
            return (pval, c_above, done)

        def run_refine(st):
            pval, c_above, done, lvl = st
            clear(NBINS // L)
            psh = jnp.where(lvl == 1, 20, 8)
            bsh = jnp.where(lvl == 1, 8, 0)
            pref = pval >> psh

            def hb(i):
                key = _mono(lax.bitcast_convert_type(data[pl.ds(i * L, L)],
                                                     jnp.int32))
                inp = (key >> psh) == pref
                bucket = (key >> bsh) & 0xFFF
                plsc.addupdate_scatter(hist, (bucket,), ones, mask=inp)
            plsc.parallel_loop(0, NV, 1, unroll=16)(hb)
            b, c_above, n_b = find_bin(c_above, NBINS // L)
            pval = pval | (b << bsh)
            done = (c_above + n_b) <= CAP_SAFE
            return (pval, c_above, done, lvl + 1)

        st0 = (jnp.int32(0), jnp.int32(0), jnp.bool_(False))
        st0 = run_level0(st0)
        st = (st0[0], st0[1], st0[2], jnp.int32(1))
        st = lax.while_loop(lambda q: (~q[2]) & (q[3] <= 2), run_refine, st)
        pval, c_above, done, _lvl = st

        def collect_a(_):
            def cb(i, off):
                key = _mono(lax.bitcast_convert_type(data[pl.ds(i * L, L)],
                                                     jnp.int32))
                m = key >= pval
                offc = jnp.minimum(off[0], CAP - L)
                plsc.store_compressed(cpos.at[pl.ds(offc, L)], i * L + lanes,
                                      mask=m)
                return off + plsc.all_reduce_population_count(m)
            off = plsc.parallel_loop(0, NV, 1, unroll=8,
                                     carry=jnp.zeros((L,), jnp.int32))(cb)
            return off[0]

        def collect_b(_):
            def cb(i, st2):
                offa, offb = st2
                key = _mono(lax.bitcast_convert_type(data[pl.ds(i * L, L)],
                                                     jnp.int32))
                mg = key > pval
                me = key == pval
                oa = jnp.minimum(offa, 80)
                ob = jnp.minimum(offb, EQCAP)
                plsc.store_compressed(ckeys.at[pl.ds(oa, L)], key, mask=mg)
                plsc.store_compressed(cpos.at[pl.ds(oa, L)], i * L + lanes,
                                      mask=mg)
                plsc.store_compressed(ekeys.at[pl.ds(ob, L)], key, mask=me)
                plsc.store_compressed(epos.at[pl.ds(ob, L)], i * L + lanes,
                                      mask=me)
                return (offa + jnp.sum(mg.astype(jnp.int32)),
                        offb + jnp.sum(me.astype(jnp.int32)))
            offa, offb = lax.fori_loop(0, NV, cb,
                                       (jnp.int32(0), jnp.int32(0)))
            nb_eq = jnp.minimum(offb, jnp.int32(EQCAP))

            def ab(c, _):
                ke = ekeys[pl.ds(c * L, L)]
                pe = epos[pl.ds(c * L, L)]
                valid = (c * L + lanes) < nb_eq
                ckeys[pl.ds(offa + c * L, L)] = jnp.where(valid, ke, minvec)
                cpos[pl.ds(offa + c * L, L)] = pe
                return 0
            lax.fori_loop(0, EQCAP // L, ab, 0)
            return offa + nb_eq

        m_cnt = lax.cond(done, collect_a, collect_b, 0)

        def kg(c):
            valid = (c * L + lanes) < m_cnt
            pos_c = cpos[pl.ds(c * L, L)]
            kv = plsc.load_gather(data, (pos_c,), mask=valid)
            key_c = _mono(lax.bitcast_convert_type(kv, jnp.int32))
            ckeys[pl.ds(c * L, L)] = jnp.where(valid, key_c, minvec)
        plsc.parallel_loop(0, CAP // L, 1, unroll=4)(kg)

        nrow = jnp.minimum(row + 1, base_row + RPW - 1)
        pltpu.async_copy(lp_hbm.at[nrow // BEAM, nrow % BEAM], data, sem0)

        def bmax(c, cm):
            m = jnp.max(ckeys[pl.ds(c * L, L)])
            return jnp.where(lanes == c, m, cm)
        CM = lax.fori_loop(0, CAP // L, bmax, minvec)

        def sel(t, cm):
            gmax = jnp.max(cm)
            csp = plsc.all_reduce_ffs(cm == gmax)
            ck = plsc.load_gather(ckeys, (csp * L + lanes,))
            lsp = plsc.all_reduce_ffs(ck == gmax)
            sel_idx = csp * L + lsp
            pos = plsc.load_gather(cpos, (sel_idx,))
            _store1(skeys, t, gmax, lanes)
            _store1(spos, t, pos, lanes)
            plsc.store_scatter(ckeys, (sel_idx,), minvec, mask=(lanes == 0))
            ck2 = jnp.where(lanes == lsp, minvec, ck)
            return jnp.where(lanes == csp, jnp.max(ck2), cm)
        lax.fori_loop(0, K, sel, CM)

        sv0 = _inv(skeys[pl.ds(0, L)])
        sv1 = _inv(skeys[pl.ds(L, L)])
        sv2 = _inv(skeys[pl.ds(2 * L, L)])
        sv3 = _inv(skeys[pl.ds(3 * L, L)])
        s0 = lax.bitcast_convert_type(sv0, jnp.float32) + gbuf[pl.ds(0, L)]
        s1 = lax.bitcast_convert_type(sv1, jnp.float32) + gbuf[pl.ds(L, L)]
        s2 = lax.bitcast_convert_type(sv2, jnp.float32) + gbuf[pl.ds(2 * L, L)]
        s3 = lax.bitcast_convert_type(sv3, jnp.float32) + gbuf[pl.ds(3 * L, L)]
        m0, m1, m2, m3 = (jnp.max(s0), jnp.max(s1), jnp.max(s2), jnp.max(s3))
        gm = jnp.maximum(jnp.maximum(m0, m1), jnp.maximum(m2, m3))
        csel = jnp.where(m0 == gm, 0,
                         jnp.where(m1 == gm, 1, jnp.where(m2 == gm, 2, 3)))
        skc = skeys[pl.ds(csel * L, L)]
        scq = lax.bitcast_convert_type(_inv(skc), jnp.float32) + gbuf[pl.ds(csel * L, L)]
        lsel = _scal(plsc.all_reduce_ffs(scq == gm))
        val = jnp.sum(jnp.where(lanes == lsel,
                                lax.bitcast_convert_type(_inv(skc), jnp.float32), 0.0))
        vid = jnp.sum(jnp.where(lanes == lsel, spos[pl.ds(csel * L, L)], 0))
        _store1(obuf_s, r, val + _load1(svbuf, r), lanes)
        _store1(obuf_i, r, vid, lanes)
        return 0

    lax.fori_loop(0, RPW, row_body, 0)
    pltpu.make_async_copy(
        lp_hbm.at[base_row // BEAM, base_row % BEAM], data, sem0).wait()
    pltpu.sync_copy(obuf_s, outs_hbm.at[pl.ds(base_row, RPW)])
    pltpu.sync_copy(obuf_i, outi_hbm.at[pl.ds(base_row, RPW)])


def kernel(step, lprobs, scores):
    bsz, beam, vocab = lprobs.shape
    g = jax.random.gumbel(jax.random.key(42), (bsz * beam, K),
                          jnp.float32).reshape(-1)
    sv = lax.dynamic_index_in_dim(scores, step - 1, axis=2,
                                  keepdims=False).reshape(-1)

    mesh = plsc.VectorSubcoreMesh(core_axis_name="c", subcore_axis_name="s",
                                  num_cores=NC, num_subcores=NS)
    out_s, out_i = pl.kernel(
        _body,
        out_type=(jax.ShapeDtypeStruct((ROWS,), jnp.float32),
                  jax.ShapeDtypeStruct((ROWS,), jnp.int32)),
        mesh=mesh,
        compiler_params=pltpu.CompilerParams(needs_layout_passes=False),
        scratch_types=[
            pltpu.VMEM((V,), jnp.float32),
            pltpu.VMEM((NBINS,), jnp.int32),
            pltpu.VMEM((CAP,), jnp.int32),
            pltpu.VMEM((CAP,), jnp.int32),
            pltpu.VMEM((96,), jnp.int32),
            pltpu.VMEM((96,), jnp.int32),
            pltpu.VMEM((K,), jnp.int32),
            pltpu.VMEM((K,), jnp.int32),
            pltpu.VMEM((K,), jnp.float32),
            pltpu.VMEM((RPW,), jnp.float32),
            pltpu.VMEM((RPW,), jnp.float32),
            pltpu.VMEM((RPW,), jnp.int32),
            pltpu.SemaphoreType.DMA,
            pltpu.SemaphoreType.DMA,
        ],
    )(lprobs, g, sv)

    scores_buf = out_s.reshape(bsz, beam)
    indices_buf = out_i.reshape(bsz, beam)
    beams_buf = jnp.broadcast_to(jnp.arange(beam, dtype=jnp.int32),
                                 (bsz, beam))
    return scores_buf, indices_buf, beams_buf
